# Initial kernel scaffold; baseline (speedup 1.0000x reference)
import jax
import jax.numpy as jnp
from jax.experimental import pallas as pl


def kernel(x, mask, W_qkv, b_qkv, W_out, b_out):
    raise NotImplementedError("write your pallas kernel here")


# speedup vs baseline: 2.2570x; 2.2570x over previous
import functools

import jax
import jax.numpy as jnp
from jax.experimental import pallas as pl
from jax.experimental.pallas import tpu as pltpu

B, T, C = 2, 2048, 1024
N_HEAD = 16
HEAD_DIM = C // N_HEAD

BLK_Q = 256
ROW_BLK = 512


def _qkv_proj_kernel(x_ref, w_ref, b_ref, o_ref):
    o_ref[...] = (
        jnp.dot(x_ref[...], w_ref[...], preferred_element_type=jnp.float32)
        + b_ref[...]
    )


def _attn_kernel(q_ref, k_ref, v_ref, wo_ref, bo_ref, o_ref):
    qi = pl.program_id(1)
    scale = 1.0 / (HEAD_DIM ** 0.5)

    q = q_ref[0] * scale
    k = k_ref[0]
    v = v_ref[0]

    row_ids = qi * BLK_Q + jax.lax.broadcasted_iota(jnp.int32, (BLK_Q, T), 0)
    col_ids = jax.lax.broadcasted_iota(jnp.int32, (BLK_Q, T), 1)
    neg_mask = col_ids > row_ids

    ys = []
    for h in range(N_HEAD):
        sl = slice(h * HEAD_DIM, (h + 1) * HEAD_DIM)
        q_h = q[:, sl]
        k_h = k[:, sl]
        v_h = v[:, sl]
        s = jax.lax.dot_general(
            q_h, k_h, (((1,), (1,)), ((), ())),
            preferred_element_type=jnp.float32,
        )
        s = jnp.where(neg_mask, -1e30, s)
        m = jnp.max(s, axis=-1, keepdims=True)
        p = jnp.exp(s - m)
        l = jnp.sum(p, axis=-1, keepdims=True)
        y_h = jax.lax.dot_general(
            p, v_h, (((1,), (0,)), ((), ())),
            preferred_element_type=jnp.float32,
        )
        ys.append(y_h * (1.0 / l))
    y = jnp.concatenate(ys, axis=-1)
    o_ref[0] = (
        jnp.dot(y, wo_ref[...], preferred_element_type=jnp.float32)
        + bo_ref[...]
    )


@functools.partial(jax.jit, static_argnames=())
def kernel(x, mask, W_qkv, b_qkv, W_out, b_out):
    del mask

    x2d = x.reshape(B * T, C)
    qkv2d = pl.pallas_call(
        _qkv_proj_kernel,
        grid=(B * T // ROW_BLK,),
        in_specs=[
            pl.BlockSpec((ROW_BLK, C), lambda i: (i, 0)),
            pl.BlockSpec((C, 3 * C), lambda i: (0, 0)),
            pl.BlockSpec((1, 3 * C), lambda i: (0, 0)),
        ],
        out_specs=pl.BlockSpec((ROW_BLK, 3 * C), lambda i: (i, 0)),
        out_shape=jax.ShapeDtypeStruct((B * T, 3 * C), jnp.float32),
        compiler_params=pltpu.CompilerParams(
            dimension_semantics=("parallel",),
            vmem_limit_bytes=100 * 1024 * 1024,
        ),
    )(x2d, W_qkv, b_qkv.reshape(1, 3 * C))
    qkv = qkv2d.reshape(B, T, 3 * C)

    out = pl.pallas_call(
        _attn_kernel,
        grid=(B, T // BLK_Q),
        in_specs=[
            pl.BlockSpec((1, BLK_Q, C), lambda b, i: (b, i, 0)),
            pl.BlockSpec((1, T, C), lambda b, i: (b, 0, 1)),
            pl.BlockSpec((1, T, C), lambda b, i: (b, 0, 2)),
            pl.BlockSpec((C, C), lambda b, i: (0, 0)),
            pl.BlockSpec((1, C), lambda b, i: (0, 0)),
        ],
        out_specs=pl.BlockSpec((1, BLK_Q, C), lambda b, i: (b, i, 0)),
        out_shape=jax.ShapeDtypeStruct((B, T, C), jnp.float32),
        compiler_params=pltpu.CompilerParams(
            dimension_semantics=("parallel", "arbitrary"),
            vmem_limit_bytes=100 * 1024 * 1024,
        ),
    )(qkv, qkv, qkv, W_out, b_out.reshape(1, C))
    return out
